# baseline (speedup 1.0000x reference)
import functools

import jax
import jax.numpy as jnp
from jax import lax
from jax.experimental import pallas as pl
from jax.experimental.pallas import tpu as pltpu
from jax.experimental.pallas import tpu_sc as plsc

VOCAB_SIZE = 100000
HIDDEN = 768
MAX_POS = 512
BATCH = 1024
SEQ = 200

NUM_WORKERS = 32
SEQ_PER_WORKER = BATCH // NUM_WORKERS
N_PHASES = 2
SEQ_PER_PHASE = SEQ_PER_WORKER // N_PHASES
CH = 40
N_CHUNKS = SEQ // CH
T_STEPS = N_CHUNKS * SEQ_PER_PHASE
NBUF = 2
LANES = 16
VECS = HIDDEN // LANES


def _sc_embed(ids_flat, word_embeddings, position_embeddings):
    mesh = plsc.VectorSubcoreMesh(core_axis_name="c", subcore_axis_name="s")

    @functools.partial(
        pl.kernel,
        out_type=jax.ShapeDtypeStruct((BATCH * SEQ, HIDDEN), jnp.float32),
        mesh=mesh,
        scratch_types=[
            pltpu.VMEM((SEQ_PER_PHASE * SEQ,), jnp.int32),
            pltpu.VMEM((CH, HIDDEN), jnp.float32),
        ] + [pltpu.VMEM((CH, HIDDEN), jnp.float32)] * NBUF
          + [pltpu.SemaphoreType.DMA] * (2 * NBUF),
    )
    def k(ids_hbm, word_hbm, pos_hbm, out_hbm, idx_v, pos_v, *ring):
        rows = ring[:NBUF]
        gsem = ring[NBUF:2 * NBUF]
        ssem = ring[2 * NBUF:]
        wid = lax.axis_index("s") * 2 + lax.axis_index("c")

        def phase(h):
            row_base = (wid * SEQ_PER_WORKER + h * SEQ_PER_PHASE) * SEQ

            def chunk_addr(t):
                c = t // SEQ_PER_PHASE
                b = t - c * SEQ_PER_PHASE
                return c, b, b * SEQ + c * CH

            def start_gather(t, p):
                _, _, l = chunk_addr(t)
                pltpu.async_copy(
                    word_hbm.at[idx_v.at[pl.ds(l, CH)]], rows[p], gsem[p])

            def wait_gather(t, p):
                _, _, l = chunk_addr(t)
                pltpu.make_async_copy(
                    word_hbm.at[idx_v.at[pl.ds(l, CH)]], rows[p], gsem[p]).wait()

            def start_store(t, p):
                _, _, l = chunk_addr(t)
                pltpu.async_copy(
                    rows[p], out_hbm.at[pl.ds(row_base + l, CH)], ssem[p])

            def wait_store(t, p):
                _, _, l = chunk_addr(t)
                pltpu.make_async_copy(
                    rows[p], out_hbm.at[pl.ds(row_base + l, CH)], ssem[p]).wait()

            def step(t, p):
                c, b, _ = chunk_addr(t)

                @pl.when(b == 0)
                def _():
                    pltpu.sync_copy(pos_hbm.at[pl.ds(c * CH, CH)], pos_v)

                wait_gather(t, p)

                @pl.when(t >= 1)
                def _():
                    wait_store(t - 1, (p - 1) % NBUF)

                @pl.when(t + (NBUF - 1) < T_STEPS)
                def _():
                    start_gather(t + (NBUF - 1), (p + NBUF - 1) % NBUF)

                def add_row(r, _):
                    for j in range(VECS):
                        sl = pl.ds(j * LANES, LANES)
                        plsc.addupdate(rows[p].at[r, sl], pos_v[r, sl])
                    return 0

                lax.fori_loop(0, CH, add_row, 0)
                start_store(t, p)

            pltpu.sync_copy(
                ids_hbm.at[pl.ds(row_base, SEQ_PER_PHASE * SEQ)], idx_v)

            for i in range(NBUF - 1):
                start_gather(jnp.int32(i), i)

            n_full = T_STEPS // NBUF

            def body(t3, _):
                for k_ in range(NBUF):
                    step(t3 * NBUF + k_, k_)
                return 0

            lax.fori_loop(0, n_full, body, 0)
            for t_ in range(n_full * NBUF, T_STEPS):
                step(jnp.int32(t_), t_ % NBUF)
            wait_store(T_STEPS - 1, (T_STEPS - 1) % NBUF)

        for h in range(N_PHASES):
            phase(h)

    return k(ids_flat, word_embeddings, position_embeddings)


@jax.jit
def kernel(input_ids, word_embeddings, position_embeddings):
    ids_flat = input_ids.reshape(BATCH * SEQ).astype(jnp.int32)
    out = _sc_embed(ids_flat, word_embeddings, position_embeddings)
    return out.reshape(BATCH, SEQ, HIDDEN)


# speedup vs baseline: 1.3981x; 1.3981x over previous
import functools

import jax
import jax.numpy as jnp
from jax import lax
from jax.experimental import pallas as pl
from jax.experimental.pallas import tpu as pltpu
from jax.experimental.pallas import tpu_sc as plsc

VOCAB_SIZE = 100000
HIDDEN = 768
MAX_POS = 512
BATCH = 1024
SEQ = 200

NUM_WORKERS = 32
SEQ_PER_WORKER = BATCH // NUM_WORKERS
CH = 40
N_CHUNKS = SEQ // CH
T_STEPS = N_CHUNKS * SEQ_PER_WORKER
LANES = 16
VECS = HIDDEN // LANES


def _sc_embed(ids_flat, word_embeddings, position_embeddings):
    mesh = plsc.VectorSubcoreMesh(core_axis_name="c", subcore_axis_name="s")

    @functools.partial(
        pl.kernel,
        out_type=jax.ShapeDtypeStruct((BATCH * SEQ, HIDDEN), jnp.float32),
        mesh=mesh,
        scratch_types=[
            pltpu.VMEM((SEQ_PER_WORKER * SEQ,), jnp.int32),
            pltpu.VMEM((CH, HIDDEN), jnp.float32),
            pltpu.VMEM((CH, HIDDEN), jnp.float32),
            pltpu.VMEM((CH, HIDDEN), jnp.float32),
            pltpu.SemaphoreType.DMA,
            pltpu.SemaphoreType.DMA,
            pltpu.SemaphoreType.DMA,
            pltpu.SemaphoreType.DMA,
        ],
    )
    def k(ids_hbm, word_hbm, pos_hbm, out_hbm,
          idx_all, pos_v, rows0, rows1, gsem0, gsem1, ssem0, ssem1):
        wid = lax.axis_index("s") * 2 + lax.axis_index("c")
        seq_base = wid * SEQ_PER_WORKER
        row_base = seq_base * SEQ
        rows = (rows0, rows1)
        gsem = (gsem0, gsem1)
        ssem = (ssem0, ssem1)

        pltpu.sync_copy(ids_hbm.at[pl.ds(row_base, SEQ_PER_WORKER * SEQ)], idx_all)

        def chunk_addr(t):
            c = t // SEQ_PER_WORKER
            b = t - c * SEQ_PER_WORKER
            return c, b, b * SEQ + c * CH

        def start_gather(t, p):
            _, _, l = chunk_addr(t)
            pltpu.async_copy(word_hbm.at[idx_all.at[pl.ds(l, CH)]], rows[p], gsem[p])

        def wait_gather(t, p):
            _, _, l = chunk_addr(t)
            pltpu.make_async_copy(
                word_hbm.at[idx_all.at[pl.ds(l, CH)]], rows[p], gsem[p]).wait()

        def start_store(t, p):
            _, _, l = chunk_addr(t)
            pltpu.async_copy(rows[p], out_hbm.at[pl.ds(row_base + l, CH)], ssem[p])

        def wait_store(t, p):
            _, _, l = chunk_addr(t)
            pltpu.make_async_copy(
                rows[p], out_hbm.at[pl.ds(row_base + l, CH)], ssem[p]).wait()

        start_gather(0, 0)

        def step(t, p):
            c, b, _ = chunk_addr(t)

            @pl.when(b == 0)
            def _():
                pltpu.sync_copy(pos_hbm.at[pl.ds(c * CH, CH)], pos_v)

            wait_gather(t, p)

            @pl.when(t >= 1)
            def _():
                wait_store(t - 1, 1 - p)

            @pl.when(t + 1 < T_STEPS)
            def _():
                start_gather(t + 1, 1 - p)

            def add_row(r, _):
                for j in range(VECS):
                    sl = pl.ds(j * LANES, LANES)
                    plsc.addupdate(rows[p].at[r, sl], pos_v[r, sl])
                return 0

            lax.fori_loop(0, CH, add_row, 0)
            start_store(t, p)
            return 0

        def pair(t2, _):
            step(2 * t2, 0)
            step(2 * t2 + 1, 1)
            return 0

        lax.fori_loop(0, T_STEPS // 2, pair, 0)
        wait_store(T_STEPS - 1, 1)

    return k(ids_flat, word_embeddings, position_embeddings)


@jax.jit
def kernel(input_ids, word_embeddings, position_embeddings):
    ids_flat = input_ids.reshape(BATCH * SEQ).astype(jnp.int32)
    out = _sc_embed(ids_flat, word_embeddings, position_embeddings)
    return out.reshape(BATCH, SEQ, HIDDEN)


# speedup vs baseline: 1.4008x; 1.0019x over previous
import functools

import jax
import jax.numpy as jnp
from jax import lax
from jax.experimental import pallas as pl
from jax.experimental.pallas import tpu as pltpu
from jax.experimental.pallas import tpu_sc as plsc

VOCAB_SIZE = 100000
HIDDEN = 768
MAX_POS = 512
BATCH = 1024
SEQ = 200

NUM_WORKERS = 32
SEQ_PER_WORKER = BATCH // NUM_WORKERS
CH = 40
N_CHUNKS = SEQ // CH
T_STEPS = N_CHUNKS * SEQ_PER_WORKER
LANES = 16
VECS = HIDDEN // LANES


def _sc_embed(ids_flat, word_embeddings, position_embeddings):
    mesh = plsc.VectorSubcoreMesh(core_axis_name="c", subcore_axis_name="s")

    @functools.partial(
        pl.kernel,
        out_type=jax.ShapeDtypeStruct((BATCH * SEQ, HIDDEN), jnp.float32),
        mesh=mesh,
        scratch_types=[
            pltpu.VMEM((SEQ_PER_WORKER * SEQ,), jnp.int32),
            pltpu.VMEM((CH, HIDDEN), jnp.float32),
            pltpu.VMEM((CH, HIDDEN), jnp.float32),
            pltpu.VMEM((CH, HIDDEN), jnp.float32),
            pltpu.VMEM((CH, HIDDEN), jnp.float32),
            pltpu.SemaphoreType.DMA,
            pltpu.SemaphoreType.DMA,
            pltpu.SemaphoreType.DMA,
            pltpu.SemaphoreType.DMA,
            pltpu.SemaphoreType.DMA,
            pltpu.SemaphoreType.DMA,
        ],
    )
    def k(ids_hbm, word_hbm, pos_hbm, out_hbm,
          idx_all, pos_v, rows0, rows1, rows2,
          gsem0, gsem1, gsem2, ssem0, ssem1, ssem2):
        wid = lax.axis_index("s") * 2 + lax.axis_index("c")
        seq_base = wid * SEQ_PER_WORKER
        row_base = seq_base * SEQ
        rows = (rows0, rows1, rows2)
        gsem = (gsem0, gsem1, gsem2)
        ssem = (ssem0, ssem1, ssem2)

        pltpu.sync_copy(ids_hbm.at[pl.ds(row_base, SEQ_PER_WORKER * SEQ)], idx_all)

        def chunk_addr(t):
            c = t // SEQ_PER_WORKER
            b = t - c * SEQ_PER_WORKER
            return c, b, b * SEQ + c * CH

        def start_gather(t, p):
            _, _, l = chunk_addr(t)
            pltpu.async_copy(word_hbm.at[idx_all.at[pl.ds(l, CH)]], rows[p], gsem[p])

        def wait_gather(t, p):
            _, _, l = chunk_addr(t)
            pltpu.make_async_copy(
                word_hbm.at[idx_all.at[pl.ds(l, CH)]], rows[p], gsem[p]).wait()

        def start_store(t, p):
            _, _, l = chunk_addr(t)
            pltpu.async_copy(rows[p], out_hbm.at[pl.ds(row_base + l, CH)], ssem[p])

        def wait_store(t, p):
            _, _, l = chunk_addr(t)
            pltpu.make_async_copy(
                rows[p], out_hbm.at[pl.ds(row_base + l, CH)], ssem[p]).wait()

        start_gather(0, 0)
        start_gather(1, 1)

        def step(t, p):
            c, b, _ = chunk_addr(t)

            @pl.when(b == 0)
            def _():
                pltpu.sync_copy(pos_hbm.at[pl.ds(c * CH, CH)], pos_v)

            wait_gather(t, p)

            @pl.when(t >= 1)
            def _():
                wait_store(t - 1, (p - 1) % 3)

            @pl.when(t + 2 < T_STEPS)
            def _():
                start_gather(t + 2, (p + 2) % 3)

            def add_row(r, _):
                for j in range(VECS):
                    sl = pl.ds(j * LANES, LANES)
                    plsc.addupdate(rows[p].at[r, sl], pos_v[r, sl])
                return 0

            lax.fori_loop(0, CH, add_row, 0)
            start_store(t, p)
            return 0

        def trip(t3, _):
            step(3 * t3, 0)
            step(3 * t3 + 1, 1)
            step(3 * t3 + 2, 2)
            return 0

        lax.fori_loop(0, T_STEPS // 3, trip, 0)
        step(jnp.int32(T_STEPS - 1), (T_STEPS - 1) % 3)
        wait_store(T_STEPS - 1, (T_STEPS - 1) % 3)

    return k(ids_flat, word_embeddings, position_embeddings)


@jax.jit
def kernel(input_ids, word_embeddings, position_embeddings):
    ids_flat = input_ids.reshape(BATCH * SEQ).astype(jnp.int32)
    out = _sc_embed(ids_flat, word_embeddings, position_embeddings)
    return out.reshape(BATCH, SEQ, HIDDEN)
